# baseline (speedup 1.0000x reference)
import functools

import jax
import jax.numpy as jnp
from jax import lax
from jax.experimental import pallas as pl
from jax.experimental.pallas import tpu as pltpu
from jax.experimental.pallas import tpu_sc as plsc

BATCH = 16384
DIM = 64
NUM_WORKERS = 32
B_PER_W = BATCH // NUM_WORKERS
CHUNK = 128
CHUNKS_PER_W = B_PER_W // CHUNK
GROUPS = B_PER_W // 16


def _sc_body(uidx_hbm, iidx_hbm, uf_hbm, vf_hbm, ub_hbm, vb_hbm, gb_hbm,
             out_hbm, idx_u, idx_i, uf_v, vf_v, ub_v, vb_v, out_v, gb_v, sem):
    wid = lax.axis_index("s") * 2 + lax.axis_index("c")
    chunk_base = wid * CHUNKS_PER_W

    pltpu.sync_copy(uidx_hbm.at[pl.ds(chunk_base, CHUNKS_PER_W), :], idx_u)
    pltpu.sync_copy(iidx_hbm.at[pl.ds(chunk_base, CHUNKS_PER_W), :], idx_i)
    pltpu.sync_copy(gb_hbm, gb_v)

    copies = []
    for ch in range(CHUNKS_PER_W):
        rows_sl = pl.ds(ch * CHUNK, CHUNK)
        copies.append(pltpu.async_copy(uf_hbm.at[idx_u.at[ch]],
                                       uf_v.at[rows_sl, :], sem))
        copies.append(pltpu.async_copy(vf_hbm.at[idx_i.at[ch]],
                                       vf_v.at[rows_sl, :], sem))
        copies.append(pltpu.async_copy(ub_hbm.at[idx_u.at[ch]],
                                       ub_v.at[rows_sl], sem))
        copies.append(pltpu.async_copy(vb_hbm.at[idx_i.at[ch]],
                                       vb_v.at[rows_sl], sem))
    for c in copies:
        c.wait()

    lanes = lax.iota(jnp.int32, 16)
    zeros16 = jnp.zeros((16,), jnp.int32)
    gb16 = gb_v[...]

    def group_body(g, carry):
        rows = g * 16 + lanes
        acc = jnp.zeros((16,), jnp.float32)
        for d in range(DIM):
            cols = jnp.full((16,), d, jnp.int32)
            acc = acc + (plsc.load_gather(uf_v, [rows, cols]) *
                         plsc.load_gather(vf_v, [rows, cols]))
        ub16 = ub_v[pl.ds(g * 16, 16)]
        vb16 = vb_v[pl.ds(g * 16, 16)]
        out_v[pl.ds(g * 16, 16)] = acc + ub16 + vb16 + gb16
        return carry

    lax.fori_loop(0, GROUPS, group_body, 0)

    pltpu.sync_copy(out_v, out_hbm.at[pl.ds(wid * B_PER_W, B_PER_W)])


@functools.partial(
    pl.kernel,
    out_type=jax.ShapeDtypeStruct((BATCH,), jnp.float32),
    mesh=plsc.VectorSubcoreMesh(core_axis_name="c", subcore_axis_name="s"),
    scratch_types=[
        pltpu.VMEM((CHUNKS_PER_W, CHUNK), jnp.int32),
        pltpu.VMEM((CHUNKS_PER_W, CHUNK), jnp.int32),
        pltpu.VMEM((B_PER_W, DIM), jnp.float32),
        pltpu.VMEM((B_PER_W, DIM), jnp.float32),
        pltpu.VMEM((B_PER_W,), jnp.float32),
        pltpu.VMEM((B_PER_W,), jnp.float32),
        pltpu.VMEM((B_PER_W,), jnp.float32),
        pltpu.VMEM((16,), jnp.float32),
        pltpu.SemaphoreType.DMA,
    ],
    compiler_params=pltpu.CompilerParams(needs_layout_passes=False,
                                         use_tc_tiling_on_sc=False),
)
def _sc_kernel(*refs):
    _sc_body(*refs)


def kernel(user_idx, item_idx, user_factors, item_factors, user_bias,
           item_bias, global_bias):
    uidx = user_idx.astype(jnp.int32).reshape(BATCH // CHUNK, CHUNK)
    iidx = item_idx.astype(jnp.int32).reshape(BATCH // CHUNK, CHUNK)
    gb = jnp.broadcast_to(global_bias, (16,))
    return _sc_kernel(uidx, iidx, user_factors, item_factors,
                      user_bias.reshape(-1), item_bias.reshape(-1), gb)


# speedup vs baseline: 1.0554x; 1.0554x over previous
import functools

import jax
import jax.numpy as jnp
from jax import lax
from jax.experimental import pallas as pl
from jax.experimental.pallas import tpu as pltpu
from jax.experimental.pallas import tpu_sc as plsc

BATCH = 16384
DIM = 64
PAD_DIM = 128
NUM_WORKERS = 32
B_PER_W = BATCH // NUM_WORKERS
CHUNK = 128
CHUNKS_PER_W = B_PER_W // CHUNK
GROUPS_PER_CHUNK = CHUNK // 16


def _sc_body(uidx_hbm, iidx_hbm, uf_hbm, vf_hbm, ub_hbm, vb_hbm, gb_hbm,
             out_hbm, idx_u, idx_i, uf_v, vf_v, ub_v, vb_v, out_v, gb_v,
             sem, bsem):
    wid = lax.axis_index("s") * 2 + lax.axis_index("c")
    chunk_base = wid * CHUNKS_PER_W

    pltpu.sync_copy(uidx_hbm.at[pl.ds(chunk_base, CHUNKS_PER_W), :], idx_u)
    pltpu.sync_copy(iidx_hbm.at[pl.ds(chunk_base, CHUNKS_PER_W), :], idx_i)
    pltpu.sync_copy(gb_hbm, gb_v)

    bias_copies = []
    for ch in range(CHUNKS_PER_W):
        rows_sl = pl.ds(ch * CHUNK, CHUNK)
        bias_copies.append(pltpu.async_copy(ub_hbm.at[idx_u.at[ch]],
                                            ub_v.at[rows_sl], bsem))
        bias_copies.append(pltpu.async_copy(vb_hbm.at[idx_i.at[ch]],
                                            vb_v.at[rows_sl], bsem))
    for c in bias_copies:
        c.wait()

    lanes = lax.iota(jnp.int32, 16)
    gb16 = gb_v[...]

    for ch in range(CHUNKS_PER_W):
        cu = pltpu.async_copy(uf_hbm.at[idx_u.at[ch]], uf_v, sem)
        cv = pltpu.async_copy(vf_hbm.at[idx_i.at[ch]], vf_v, sem)
        cu.wait()
        cv.wait()

        def group_body(g, carry, _ch=ch):
            rows = g * 16 + lanes
            acc = jnp.zeros((16,), jnp.float32)
            for d in range(DIM):
                cols = jnp.full((16,), d, jnp.int32)
                acc = acc + (plsc.load_gather(uf_v, [rows, cols]) *
                             plsc.load_gather(vf_v, [rows, cols]))
            base = _ch * CHUNK
            ub16 = ub_v[pl.ds(base + g * 16, 16)]
            vb16 = vb_v[pl.ds(base + g * 16, 16)]
            out_v[pl.ds(base + g * 16, 16)] = acc + ub16 + vb16 + gb16
            return carry

        lax.fori_loop(0, GROUPS_PER_CHUNK, group_body, 0)

    pltpu.sync_copy(out_v, out_hbm.at[pl.ds(wid * B_PER_W, B_PER_W)])


@functools.partial(
    pl.kernel,
    out_type=jax.ShapeDtypeStruct((BATCH,), jnp.float32),
    mesh=plsc.VectorSubcoreMesh(core_axis_name="c", subcore_axis_name="s"),
    scratch_types=[
        pltpu.VMEM((CHUNKS_PER_W, CHUNK), jnp.int32),
        pltpu.VMEM((CHUNKS_PER_W, CHUNK), jnp.int32),
        pltpu.VMEM((CHUNK, PAD_DIM), jnp.float32),
        pltpu.VMEM((CHUNK, PAD_DIM), jnp.float32),
        pltpu.VMEM((B_PER_W,), jnp.float32),
        pltpu.VMEM((B_PER_W,), jnp.float32),
        pltpu.VMEM((B_PER_W,), jnp.float32),
        pltpu.VMEM((16,), jnp.float32),
        pltpu.SemaphoreType.DMA,
        pltpu.SemaphoreType.DMA,
    ],
    compiler_params=pltpu.CompilerParams(needs_layout_passes=False,
                                         use_tc_tiling_on_sc=True),
)
def _sc_kernel(*refs):
    _sc_body(*refs)


def kernel(user_idx, item_idx, user_factors, item_factors, user_bias,
           item_bias, global_bias):
    uidx = user_idx.astype(jnp.int32).reshape(BATCH // CHUNK, CHUNK)
    iidx = item_idx.astype(jnp.int32).reshape(BATCH // CHUNK, CHUNK)
    ufp = jnp.pad(user_factors, ((0, 0), (0, PAD_DIM - DIM)))
    vfp = jnp.pad(item_factors, ((0, 0), (0, PAD_DIM - DIM)))
    gb = jnp.broadcast_to(global_bias, (16,))
    return _sc_kernel(uidx, iidx, ufp, vfp,
                      user_bias.reshape(-1), item_bias.reshape(-1), gb)
